# baseline (speedup 1.0000x reference)
import functools

import jax
import jax.numpy as jnp
from jax import lax
from jax.experimental import pallas as pl
from jax.experimental.pallas import tpu as pltpu
from jax.experimental.pallas import tpu_sc as plsc

HIDDEN = 768
EPS = 1e-12

_INFO = plsc.get_sparse_core_info()
_NC = _INFO.num_cores
_NS = _INFO.num_subcores
_NW = _NC * _NS

_B, _S = 4, 2048
_TOKENS = _B * _S
_Q = 2


def _sc_gather(input_ids, token_table, piece):
    mesh = plsc.VectorSubcoreMesh(core_axis_name="c", subcore_axis_name="s")
    C = _S // _Q
    n_tokens = _B * C
    tok_per_w = n_tokens // _NW
    w_per_row = C // tok_per_w
    ch = min(tok_per_w, 128)
    nch = tok_per_w // ch

    @functools.partial(
        pl.kernel,
        mesh=mesh,
        out_type=jax.ShapeDtypeStruct((n_tokens, HIDDEN), jnp.float32),
        scratch_types=[
            pltpu.VMEM((tok_per_w,), jnp.int32),
            pltpu.VMEM((ch, HIDDEN), jnp.float32),
            pltpu.VMEM((ch, HIDDEN), jnp.float32),
            pltpu.SemaphoreType.DMA,
            pltpu.SemaphoreType.DMA,
            pltpu.SemaphoreType.DMA,
            pltpu.SemaphoreType.DMA,
        ],
    )
    def k(ids_ref, table_ref, out_ref, idx_v, buf0, buf1,
          sem0, sem1, wsem0, wsem1):
        wid = lax.axis_index("s") * _NC + lax.axis_index("c")
        base = wid * tok_per_w
        row = wid // w_per_row
        col = piece * C + (wid % w_per_row) * tok_per_w
        pltpu.sync_copy(ids_ref.at[row, pl.ds(col, tok_per_w)], idx_v)
        bufs = (buf0, buf1)
        gsems = (sem0, sem1)
        wsems = (wsem0, wsem1)
        gcps = [pltpu.async_copy(
                    table_ref.at[idx_v.at[pl.ds(c * ch, ch)]],
                    bufs[c], gsems[c])
                for c in range(nch)]
        wcps = []
        for c in range(nch):
            gcps[c].wait()
            wcps.append(pltpu.async_copy(
                bufs[c], out_ref.at[pl.ds(base + c * ch, ch)], wsems[c]))
        for w in wcps:
            w.wait()

    return k(input_ids, token_table)


def _tc_ln_body(*refs):
    g_ref, p_ref, gamma_ref, beta_ref = refs[:4]
    o_ref = refs[-1]
    x = g_ref[...] + p_ref[...]
    mean = jnp.mean(x, axis=-1, keepdims=True)
    xc = x - mean
    var = jnp.mean(xc * xc, axis=-1, keepdims=True)
    o_ref[...] = ((xc * lax.rsqrt(var + EPS)) * gamma_ref[...][None, :]
                  + beta_ref[...][None, :])


def _tc_ln_into(gathered, pos_table, gamma, beta, dst, piece):
    C = _S // _Q
    in_specs = [
        pl.BlockSpec((C, HIDDEN), lambda j: (j, 0)),
        pl.BlockSpec((C, HIDDEN), lambda j: (piece, 0)),
        pl.BlockSpec((HIDDEN,), lambda j: (0,)),
        pl.BlockSpec((HIDDEN,), lambda j: (0,)),
    ]
    args = [gathered, pos_table, gamma, beta]
    aliases = {}
    if dst is not None:
        in_specs.append(pl.BlockSpec(memory_space=pltpu.MemorySpace.HBM))
        args.append(dst)
        aliases = {4: 0}
    return pl.pallas_call(
        _tc_ln_body,
        grid=(_B,),
        in_specs=in_specs,
        out_specs=pl.BlockSpec((C, HIDDEN), lambda j: (j * _Q + piece, 0)),
        out_shape=jax.ShapeDtypeStruct((_TOKENS, HIDDEN), jnp.float32),
        input_output_aliases=aliases,
    )(*args)


def kernel(input_ids, token_table, pos_table, gamma, beta):
    B, S = input_ids.shape
    g = [_sc_gather(input_ids, token_table, h) for h in range(_Q)]
    dst = None
    for h in range(_Q):
        dst = _tc_ln_into(g[h], pos_table, gamma, beta, dst, h)
    return dst.reshape(B, S, HIDDEN)


# speedup vs baseline: 1.0012x; 1.0012x over previous
import functools

import jax
import jax.numpy as jnp
from jax import lax
from jax.experimental import pallas as pl
from jax.experimental.pallas import tpu as pltpu
from jax.experimental.pallas import tpu_sc as plsc

HIDDEN = 768
EPS = 1e-12

_INFO = plsc.get_sparse_core_info()
_NC = _INFO.num_cores
_NS = _INFO.num_subcores
_NW = _NC * _NS

_B, _S = 4, 2048
_TOKENS = _B * _S
_Q = 2


def _sc_gather(input_ids, token_table, piece):
    mesh = plsc.VectorSubcoreMesh(core_axis_name="c", subcore_axis_name="s")
    C = _S // _Q
    n_tokens = _B * C
    tok_per_w = n_tokens // _NW
    w_per_row = C // tok_per_w
    ch = min(tok_per_w, 64)
    nch = tok_per_w // ch

    @functools.partial(
        pl.kernel,
        mesh=mesh,
        out_type=jax.ShapeDtypeStruct((n_tokens, HIDDEN), jnp.float32),
        scratch_types=[
            pltpu.VMEM((tok_per_w,), jnp.int32),
            pltpu.VMEM((ch, HIDDEN), jnp.float32),
            pltpu.VMEM((ch, HIDDEN), jnp.float32),
            pltpu.SemaphoreType.DMA,
            pltpu.SemaphoreType.DMA,
            pltpu.SemaphoreType.DMA,
            pltpu.SemaphoreType.DMA,
        ],
    )
    def k(ids_ref, table_ref, out_ref, idx_v, buf0, buf1,
          sem0, sem1, wsem0, wsem1):
        wid = lax.axis_index("s") * _NC + lax.axis_index("c")
        base = wid * tok_per_w
        row = wid // w_per_row
        col = piece * C + (wid % w_per_row) * tok_per_w
        pltpu.sync_copy(ids_ref.at[row, pl.ds(col, tok_per_w)], idx_v)
        bufs = (buf0, buf1)
        gsems = (sem0, sem1)
        wsems = (wsem0, wsem1)
        gcps = [pltpu.async_copy(
                    table_ref.at[idx_v.at[pl.ds(c * ch, ch)]],
                    bufs[c], gsems[c])
                for c in range(nch)]
        wcps = []
        for c in range(nch):
            gcps[c].wait()
            wcps.append(pltpu.async_copy(
                bufs[c], out_ref.at[pl.ds(base + c * ch, ch)], wsems[c]))
        for w in wcps:
            w.wait()

    return k(input_ids, token_table)


def _tc_ln_body(*refs):
    g_ref, p_ref, gamma_ref, beta_ref = refs[:4]
    o_ref = refs[-1]
    x = g_ref[...] + p_ref[...]
    mean = jnp.mean(x, axis=-1, keepdims=True)
    xc = x - mean
    var = jnp.mean(xc * xc, axis=-1, keepdims=True)
    o_ref[...] = ((xc * lax.rsqrt(var + EPS)) * gamma_ref[...][None, :]
                  + beta_ref[...][None, :])


def _tc_ln_into(gathered, pos_table, gamma, beta, dst, piece):
    C = _S // _Q
    in_specs = [
        pl.BlockSpec((C, HIDDEN), lambda j: (j, 0)),
        pl.BlockSpec((C, HIDDEN), lambda j: (piece, 0)),
        pl.BlockSpec((HIDDEN,), lambda j: (0,)),
        pl.BlockSpec((HIDDEN,), lambda j: (0,)),
    ]
    args = [gathered, pos_table, gamma, beta]
    aliases = {}
    if dst is not None:
        in_specs.append(pl.BlockSpec(memory_space=pltpu.MemorySpace.HBM))
        args.append(dst)
        aliases = {4: 0}
    return pl.pallas_call(
        _tc_ln_body,
        grid=(_B,),
        in_specs=in_specs,
        out_specs=pl.BlockSpec((C, HIDDEN), lambda j: (j * _Q + piece, 0)),
        out_shape=jax.ShapeDtypeStruct((_TOKENS, HIDDEN), jnp.float32),
        input_output_aliases=aliases,
    )(*args)


def kernel(input_ids, token_table, pos_table, gamma, beta):
    B, S = input_ids.shape
    g = [_sc_gather(input_ids, token_table, h) for h in range(_Q)]
    dst = None
    for h in range(_Q):
        dst = _tc_ln_into(g[h], pos_table, gamma, beta, dst, h)
    return dst.reshape(B, S, HIDDEN)


# speedup vs baseline: 1.0169x; 1.0156x over previous
import functools

import jax
import jax.numpy as jnp
from jax import lax
from jax.experimental import pallas as pl
from jax.experimental.pallas import tpu as pltpu
from jax.experimental.pallas import tpu_sc as plsc

HIDDEN = 768
EPS = 1e-12

_INFO = plsc.get_sparse_core_info()
_NC = _INFO.num_cores
_NS = _INFO.num_subcores
_NW = _NC * _NS

_B, _S = 4, 2048
_TOKENS = _B * _S
_HALVES = 2
_BH = _B // _HALVES
_NTOK = _BH * _S
_TOK_PER_W = _NTOK // _NW
_CH = 64
_NCH = _TOK_PER_W // _CH


def _sc_gather(input_ids, token_table, half):
    mesh = plsc.VectorSubcoreMesh(core_axis_name="c", subcore_axis_name="s")
    w_per_seq = _S // _TOK_PER_W

    @functools.partial(
        pl.kernel,
        mesh=mesh,
        out_type=jax.ShapeDtypeStruct((_NTOK, HIDDEN), jnp.float32),
        scratch_types=[
            pltpu.VMEM((_TOK_PER_W,), jnp.int32),
            pltpu.VMEM((_CH, HIDDEN), jnp.float32),
            pltpu.VMEM((_CH, HIDDEN), jnp.float32),
            pltpu.SemaphoreType.DMA,
            pltpu.SemaphoreType.DMA,
            pltpu.SemaphoreType.DMA,
            pltpu.SemaphoreType.DMA,
        ],
    )
    def k(ids_ref, table_ref, out_ref, idx_v, buf0, buf1,
          sem0, sem1, wsem0, wsem1):
        wid = lax.axis_index("s") * _NC + lax.axis_index("c")
        base = wid * _TOK_PER_W
        row = half * _BH + wid // w_per_seq
        col = (wid % w_per_seq) * _TOK_PER_W
        pltpu.sync_copy(ids_ref.at[row, pl.ds(col, _TOK_PER_W)], idx_v)
        bufs = (buf0, buf1)
        gsems = (sem0, sem1)
        wsems = (wsem0, wsem1)
        gcps = [pltpu.async_copy(
                    table_ref.at[idx_v.at[pl.ds(c * _CH, _CH)]],
                    bufs[c], gsems[c])
                for c in range(_NCH)]
        wcps = []
        for c in range(_NCH):
            gcps[c].wait()
            wcps.append(pltpu.async_copy(
                bufs[c], out_ref.at[pl.ds(base + c * _CH, _CH)], wsems[c]))
        for w in wcps:
            w.wait()

    return k(input_ids, token_table)


def _tc_ln_body(*refs):
    g_ref, p_ref, gamma_ref, beta_ref = refs[:4]
    o_ref = refs[-1]
    x = g_ref[...] + p_ref[...]
    mean = jnp.mean(x, axis=-1, keepdims=True)
    xc = x - mean
    var = jnp.mean(xc * xc, axis=-1, keepdims=True)
    o_ref[...] = ((xc * lax.rsqrt(var + EPS)) * gamma_ref[...][None, :]
                  + beta_ref[...][None, :])


def _tc_ln_into(gathered, pos_table, gamma, beta, dst, half):
    in_specs = [
        pl.BlockSpec((_S, HIDDEN), lambda j: (j, 0)),
        pl.BlockSpec((_S, HIDDEN), lambda j: (0, 0)),
        pl.BlockSpec((HIDDEN,), lambda j: (0,)),
        pl.BlockSpec((HIDDEN,), lambda j: (0,)),
    ]
    args = [gathered, pos_table, gamma, beta]
    aliases = {}
    if dst is not None:
        in_specs.append(pl.BlockSpec(memory_space=pltpu.MemorySpace.HBM))
        args.append(dst)
        aliases = {4: 0}
    return pl.pallas_call(
        _tc_ln_body,
        grid=(_BH,),
        in_specs=in_specs,
        out_specs=pl.BlockSpec((_S, HIDDEN),
                               lambda j: (half * _BH + j, 0)),
        out_shape=jax.ShapeDtypeStruct((_TOKENS, HIDDEN), jnp.float32),
        input_output_aliases=aliases,
    )(*args)


def kernel(input_ids, token_table, pos_table, gamma, beta):
    B, S = input_ids.shape
    g = [_sc_gather(input_ids, token_table, h) for h in range(_HALVES)]
    dst = None
    for h in range(_HALVES):
        dst = _tc_ln_into(g[h], pos_table, gamma, beta, dst, h)
    return dst.reshape(B, S, HIDDEN)
